# baseline (speedup 1.0000x reference)
import functools

import jax
import jax.numpy as jnp
from jax import lax
from jax.experimental import pallas as pl
from jax.experimental.pallas import tpu as pltpu
from jax.experimental.pallas import tpu_sc as plsc

_BATCH = 16384
_K = 32
_NC = 2
_NS = 16
_NW = _NC * _NS
_BPW = _BATCH // _NW
_CHUNK = 128
_NCHUNK = _BPW // _CHUNK
_LANES = 16


def _dot_body(uidx_hbm, iidx_hbm, wt_hbm, ht_hbm, out_hbm,
              uidx_v, iidx_v, ut_v, vt_v, out_v, sem):
    cid = lax.axis_index("c")
    sid = lax.axis_index("s")
    wid = sid * _NC + cid
    base = wid * _BPW

    pltpu.sync_copy(uidx_hbm.at[pl.ds(wid * _NCHUNK, _NCHUNK)], uidx_v)
    pltpu.sync_copy(iidx_hbm.at[pl.ds(wid * _NCHUNK, _NCHUNK)], iidx_v)

    copies = []
    for j in range(_NCHUNK):
        for k in range(_K):
            copies.append(pltpu.async_copy(
                wt_hbm.at[k].at[uidx_v.at[j]],
                ut_v.at[k, pl.ds(j * _CHUNK, _CHUNK)], sem))
            copies.append(pltpu.async_copy(
                ht_hbm.at[k].at[iidx_v.at[j]],
                vt_v.at[k, pl.ds(j * _CHUNK, _CHUNK)], sem))
    for c in copies:
        c.wait()

    def group(g, carry):
        acc = jnp.zeros((16,), jnp.float32)
        for k in range(_K):
            acc = acc + (ut_v[k, pl.ds(g * _LANES, _LANES)]
                         * vt_v[k, pl.ds(g * _LANES, _LANES)])
        out_v[pl.ds(g * _LANES, _LANES)] = acc
        return carry

    lax.fori_loop(0, _BPW // _LANES, group, 0)

    pltpu.sync_copy(out_v, out_hbm.at[pl.ds(base, _BPW)])


@jax.jit
def _mf_dot(uidx, iidx, wt, ht):
    mesh = plsc.VectorSubcoreMesh(core_axis_name="c", subcore_axis_name="s")
    kfn = functools.partial(
        pl.kernel,
        mesh=mesh,
        compiler_params=pltpu.CompilerParams(
            needs_layout_passes=False, use_tc_tiling_on_sc=False),
        out_type=jax.ShapeDtypeStruct((_BATCH,), jnp.float32),
        scratch_types=[
            pltpu.VMEM((_NCHUNK, _CHUNK), jnp.int32),
            pltpu.VMEM((_NCHUNK, _CHUNK), jnp.int32),
            pltpu.VMEM((_K, _BPW), jnp.float32),
            pltpu.VMEM((_K, _BPW), jnp.float32),
            pltpu.VMEM((_BPW,), jnp.float32),
            pltpu.SemaphoreType.DMA,
        ],
    )(_dot_body)
    return kfn(uidx, iidx, wt, ht)


def kernel(x, W, H):
    uidx = x[:, 0].astype(jnp.int32).reshape(_NW * _NCHUNK, _CHUNK)
    iidx = x[:, 1].astype(jnp.int32).reshape(_NW * _NCHUNK, _CHUNK)
    return _mf_dot(uidx, iidx, W.T, H.T)


# speedup vs baseline: 4.8789x; 4.8789x over previous
import functools

import jax
import jax.numpy as jnp
from jax import lax
from jax.experimental import pallas as pl
from jax.experimental.pallas import tpu as pltpu
from jax.experimental.pallas import tpu_sc as plsc

_BATCH = 16384
_K = 32
_NC = 2
_NS = 16
_NW = _NC * _NS
_BPW = _BATCH // _NW
_CHUNK = 128
_NCHUNK = _BPW // _CHUNK
_LANES = 16


def _dot_body(uidx_hbm, iidx_hbm, w_hbm, h_hbm, out_hbm,
              uidx_v, iidx_v, urows_v, vrows_v, out_v, sem):
    cid = lax.axis_index("c")
    sid = lax.axis_index("s")
    wid = sid * _NC + cid
    base = wid * _BPW

    pltpu.sync_copy(uidx_hbm.at[pl.ds(wid * _NCHUNK, _NCHUNK)], uidx_v)
    pltpu.sync_copy(iidx_hbm.at[pl.ds(wid * _NCHUNK, _NCHUNK)], iidx_v)

    copies = []
    for j in range(_NCHUNK):
        copies.append(pltpu.async_copy(
            w_hbm.at[uidx_v.at[j]], urows_v.at[pl.ds(j * _CHUNK, _CHUNK)], sem))
        copies.append(pltpu.async_copy(
            h_hbm.at[iidx_v.at[j]], vrows_v.at[pl.ds(j * _CHUNK, _CHUNK)], sem))
    for c in copies:
        c.wait()

    lane = lax.iota(jnp.int32, 16)

    def group(g, carry):
        acc = jnp.zeros((16,), jnp.float32)
        for j in range(_LANES):
            i = g * _LANES + j
            u = urows_v[i, pl.ds(0, _K)]
            v = vrows_v[i, pl.ds(0, _K)]
            ue, uo = plsc.unpack(u, format=plsc.PackFormat.INTERLEAVED)
            ve, vo = plsc.unpack(v, format=plsc.PackFormat.INTERLEAVED)
            q = ue * ve + uo * vo
            acc = jnp.where(lane == j, jnp.sum(q), acc)
        out_v[pl.ds(g * _LANES, _LANES)] = acc
        return carry

    lax.fori_loop(0, _BPW // _LANES, group, 0)

    pltpu.sync_copy(out_v, out_hbm.at[pl.ds(base, _BPW)])


@jax.jit
def _mf_dot(uidx, iidx, w, h):
    mesh = plsc.VectorSubcoreMesh(core_axis_name="c", subcore_axis_name="s")
    kfn = functools.partial(
        pl.kernel,
        mesh=mesh,
        compiler_params=pltpu.CompilerParams(
            needs_layout_passes=False, use_tc_tiling_on_sc=False),
        out_type=jax.ShapeDtypeStruct((_BATCH,), jnp.float32),
        scratch_types=[
            pltpu.VMEM((_NCHUNK, _CHUNK), jnp.int32),
            pltpu.VMEM((_NCHUNK, _CHUNK), jnp.int32),
            pltpu.VMEM((_BPW, _K), jnp.bfloat16),
            pltpu.VMEM((_BPW, _K), jnp.bfloat16),
            pltpu.VMEM((_BPW,), jnp.float32),
            pltpu.SemaphoreType.DMA,
        ],
    )(_dot_body)
    return kfn(uidx, iidx, w, h)


def kernel(x, W, H):
    uidx = x[:, 0].astype(jnp.int32).reshape(_NW * _NCHUNK, _CHUNK)
    iidx = x[:, 1].astype(jnp.int32).reshape(_NW * _NCHUNK, _CHUNK)
    return _mf_dot(uidx, iidx,
                   W.astype(jnp.bfloat16), H.astype(jnp.bfloat16))


# speedup vs baseline: 5.7144x; 1.1713x over previous
import functools

import jax
import jax.numpy as jnp
from jax import lax
from jax.experimental import pallas as pl
from jax.experimental.pallas import tpu as pltpu
from jax.experimental.pallas import tpu_sc as plsc

_BATCH = 16384
_K = 32
_NC = 2
_NS = 16
_NW = _NC * _NS
_BPW = _BATCH // _NW
_CHUNK = 128
_NCHUNK = _BPW // _CHUNK
_LANES = 16


def _dot_body(uidx_hbm, iidx_hbm, w_hbm, h_hbm, out_hbm,
              uidx_v, iidx_v, urows_v, vrows_v, out_v, sem):
    cid = lax.axis_index("c")
    sid = lax.axis_index("s")
    wid = sid * _NC + cid
    base = wid * _BPW

    pltpu.sync_copy(uidx_hbm.at[pl.ds(wid * _NCHUNK, _NCHUNK)], uidx_v)
    pltpu.sync_copy(iidx_hbm.at[pl.ds(wid * _NCHUNK, _NCHUNK)], iidx_v)

    copies = []
    for j in range(_NCHUNK):
        copies.append(pltpu.async_copy(
            w_hbm.at[uidx_v.at[j]], urows_v.at[pl.ds(j * _CHUNK, _CHUNK)], sem))
        copies.append(pltpu.async_copy(
            h_hbm.at[iidx_v.at[j]], vrows_v.at[pl.ds(j * _CHUNK, _CHUNK)], sem))
    for c in copies:
        c.wait()

    lane = lax.iota(jnp.int32, 16)

    def group(g, carry):
        acc = jnp.zeros((16,), jnp.float32)
        for j in range(_LANES):
            i = g * _LANES + j
            u0 = urows_v[i, pl.ds(0, 16)]
            u1 = urows_v[i, pl.ds(16, 16)]
            v0 = vrows_v[i, pl.ds(0, 16)]
            v1 = vrows_v[i, pl.ds(16, 16)]
            q = u0 * v0 + u1 * v1
            acc = jnp.where(lane == j, jnp.sum(q), acc)
        out_v[pl.ds(g * _LANES, _LANES)] = acc
        return carry

    lax.fori_loop(0, _BPW // _LANES, group, 0)

    pltpu.sync_copy(out_v, out_hbm.at[pl.ds(base, _BPW)])


@jax.jit
def _mf_dot(uidx, iidx, w, h):
    mesh = plsc.VectorSubcoreMesh(core_axis_name="c", subcore_axis_name="s")
    kfn = functools.partial(
        pl.kernel,
        mesh=mesh,
        compiler_params=pltpu.CompilerParams(
            needs_layout_passes=False, use_tc_tiling_on_sc=False),
        out_type=jax.ShapeDtypeStruct((_BATCH,), jnp.float32),
        scratch_types=[
            pltpu.VMEM((_NCHUNK, _CHUNK), jnp.int32),
            pltpu.VMEM((_NCHUNK, _CHUNK), jnp.int32),
            pltpu.VMEM((_BPW, _K), jnp.float32),
            pltpu.VMEM((_BPW, _K), jnp.float32),
            pltpu.VMEM((_BPW,), jnp.float32),
            pltpu.SemaphoreType.DMA,
        ],
    )(_dot_body)
    return kfn(uidx, iidx, w, h)


def kernel(x, W, H):
    uidx = x[:, 0].astype(jnp.int32).reshape(_NW * _NCHUNK, _CHUNK)
    iidx = x[:, 1].astype(jnp.int32).reshape(_NW * _NCHUNK, _CHUNK)
    return _mf_dot(uidx, iidx, W, H)
